# baseline (speedup 1.0000x reference)
import jax
import jax.numpy as jnp
from jax.experimental import pallas as pl
from jax.experimental.pallas import tpu as pltpu

_K = 64
_D_THRESH2 = 25.0
_POWER_ITERS = 10


def _topk_gather_kernel(attn_ref, rt_ref, cent_ref, toks_out, cent_out, oh_ref):
    a = attn_ref[0]
    n_tok = a.shape[1]
    iota = jax.lax.broadcasted_iota(jnp.int32, (1, n_tok), 1)

    def body(k, a):
        m = jnp.max(a)
        idx = jnp.min(jnp.where(a == m, iota, n_tok))
        oh = iota == idx
        oh_ref[pl.ds(k, 1), :] = oh.astype(jnp.float32)
        return jnp.where(oh, -jnp.inf, a)

    jax.lax.fori_loop(0, _K, body, a)
    oh = oh_ref[...]
    toks_out[0] = jnp.dot(oh, rt_ref[0], preferred_element_type=jnp.float32)
    cent_out[0] = jnp.dot(oh, cent_ref[0], preferred_element_type=jnp.float32)


def _pdist_k(x):
    d2 = jnp.zeros((x.shape[0], x.shape[0]), jnp.float32)
    for c in range(3):
        col = x[:, c:c + 1]
        diff = col - jnp.transpose(col)
        d2 = d2 + diff * diff
    return jnp.sqrt(d2 + 1e-8)


def _triplet_kernel(anc_idx, pos_idx, neg_idx,
                    anc_t_ref, pos_t_ref, neg_t_ref,
                    anc_c_ref, pos_c_ref, neg_c_ref,
                    win_ref, bin_ref, w1_ref, b1_ref, w2_ref, b2_ref,
                    out_ref):
    del anc_idx, pos_idx, neg_idx
    k = _K
    c_dim = win_ref.shape[0]
    inv_sqrt_c = 1.0 / jnp.sqrt(jnp.float32(c_dim))
    ii = jax.lax.broadcasted_iota(jnp.int32, (k, k), 0)
    jj = jax.lax.broadcasted_iota(jnp.int32, (k, k), 1)
    jj_f = jj.astype(jnp.float32)

    anc_rt = jnp.dot(anc_t_ref[0], win_ref[...],
                     preferred_element_type=jnp.float32) + bin_ref[...]
    d_anc = _pdist_k(anc_c_ref[0])

    scores = []
    for n, (t_ref, c_ref) in enumerate(((pos_t_ref, pos_c_ref),
                                        (neg_t_ref, neg_c_ref))):
        nn_rt = jnp.dot(t_ref[0], win_ref[...],
                        preferred_element_type=jnp.float32) + bin_ref[...]
        sim = jax.lax.dot_general(
            anc_rt, nn_rt, (((1,), (1,)), ((), ())),
            preferred_element_type=jnp.float32) * inv_sqrt_c
        sim = sim - jnp.max(sim, axis=1, keepdims=True)
        e = jnp.exp(sim)
        attn = e / jnp.sum(e, axis=1, keepdims=True)
        matched = jnp.dot(attn, c_ref[0],
                          preferred_element_type=jnp.float32)
        diff = d_anc - _pdist_k(matched)
        m_adj = jnp.maximum(0.0, 1.0 - diff * diff / _D_THRESH2)
        m_adj = jnp.where(ii == jj, 0.0, m_adj)

        v = jnp.full((1, k), 1.0 / jnp.sqrt(jnp.float32(k)), jnp.float32)
        for _ in range(_POWER_ITERS):
            v = jnp.dot(v, m_adj, preferred_element_type=jnp.float32)
            v = v / (jnp.sqrt(jnp.sum(v * v)) + 1e-8)

        row = jnp.broadcast_to(v, (k, k))
        col = jnp.transpose(v)
        gt = (row > col) | ((row == col) & (jj < ii))
        rank = jnp.sum(gt.astype(jnp.float32), axis=1, keepdims=True)
        sel = (rank == jj_f).astype(jnp.float32)
        v_sorted = jnp.dot(v, sel, preferred_element_type=jnp.float32)

        h = jnp.maximum(
            jnp.dot(v_sorted, w1_ref[...],
                    preferred_element_type=jnp.float32) + b1_ref[...], 0.0)
        z = jnp.sum(h * w2_ref[...]) + b2_ref[0, 0]
        scores.append(1.0 / (1.0 + jnp.exp(-z)))

    lane = jax.lax.broadcasted_iota(jnp.int32, (1, 2), 1)
    out_ref[0] = jnp.where(lane == 0, scores[0], scores[1])


def kernel(rt, rt_cls_attn, rt_centroids, anc_indices, pos_indices,
           neg_indices, W_in, b_in, W1, b1, W2, b2):
    b_dim, n_tok, c_dim = rt.shape
    t_dim = anc_indices.shape[0]
    k = _K

    attn = jnp.swapaxes(rt_cls_attn, 1, 2)
    toks_sel, cent_sel = pl.pallas_call(
        _topk_gather_kernel,
        grid=(b_dim,),
        in_specs=[
            pl.BlockSpec((1, 1, n_tok), lambda b: (b, 0, 0)),
            pl.BlockSpec((1, n_tok, c_dim), lambda b: (b, 0, 0)),
            pl.BlockSpec((1, n_tok, 3), lambda b: (b, 0, 0)),
        ],
        out_specs=[
            pl.BlockSpec((1, k, c_dim), lambda b: (b, 0, 0)),
            pl.BlockSpec((1, k, 3), lambda b: (b, 0, 0)),
        ],
        out_shape=[
            jax.ShapeDtypeStruct((b_dim, k, c_dim), jnp.float32),
            jax.ShapeDtypeStruct((b_dim, k, 3), jnp.float32),
        ],
        scratch_shapes=[pltpu.VMEM((k, n_tok), jnp.float32)],
    )(attn, rt, rt_centroids)

    anc_i = jnp.asarray(anc_indices, jnp.int32)
    pos_i = jnp.asarray(pos_indices, jnp.int32)
    neg_i = jnp.asarray(neg_indices, jnp.int32)

    grid_spec = pltpu.PrefetchScalarGridSpec(
        num_scalar_prefetch=3,
        grid=(t_dim,),
        in_specs=[
            pl.BlockSpec((1, k, c_dim), lambda t, a, p, n: (a[t], 0, 0)),
            pl.BlockSpec((1, k, c_dim), lambda t, a, p, n: (p[t], 0, 0)),
            pl.BlockSpec((1, k, c_dim), lambda t, a, p, n: (n[t], 0, 0)),
            pl.BlockSpec((1, k, 3), lambda t, a, p, n: (a[t], 0, 0)),
            pl.BlockSpec((1, k, 3), lambda t, a, p, n: (p[t], 0, 0)),
            pl.BlockSpec((1, k, 3), lambda t, a, p, n: (n[t], 0, 0)),
            pl.BlockSpec((c_dim, c_dim), lambda t, a, p, n: (0, 0)),
            pl.BlockSpec((1, c_dim), lambda t, a, p, n: (0, 0)),
            pl.BlockSpec((k, k), lambda t, a, p, n: (0, 0)),
            pl.BlockSpec((1, k), lambda t, a, p, n: (0, 0)),
            pl.BlockSpec((1, k), lambda t, a, p, n: (0, 0)),
            pl.BlockSpec((1, 1), lambda t, a, p, n: (0, 0)),
        ],
        out_specs=pl.BlockSpec((1, 1, 2), lambda t, a, p, n: (t, 0, 0)),
    )
    scores = pl.pallas_call(
        _triplet_kernel,
        grid_spec=grid_spec,
        out_shape=jax.ShapeDtypeStruct((t_dim, 1, 2), jnp.float32),
    )(anc_i, pos_i, neg_i,
      toks_sel, toks_sel, toks_sel, cent_sel, cent_sel, cent_sel,
      W_in, b_in.reshape(1, c_dim), W1, b1.reshape(1, k),
      W2.reshape(1, k), b2.reshape(1, 1))

    rerank_scores = scores.reshape(t_dim, 2, 1)
    targets = jnp.zeros_like(rerank_scores).at[:, 0].set(1.0)
    return rerank_scores, targets


# speedup vs baseline: 1.1057x; 1.1057x over previous
import jax
import jax.numpy as jnp
from jax.experimental import pallas as pl
from jax.experimental.pallas import tpu as pltpu

_K = 64
_D_THRESH2 = 25.0
_POWER_ITERS = 10


def _topk_gather_kernel(attn_ref, rt_ref, cent_ref, toks_out, cent_out):
    a = attn_ref[0]
    r_dim, l_dim = a.shape
    n_tok = r_dim * l_dim
    iota = (jax.lax.broadcasted_iota(jnp.int32, (r_dim, l_dim), 0) * l_dim
            + jax.lax.broadcasted_iota(jnp.int32, (r_dim, l_dim), 1))

    def body(k, a):
        m = jnp.max(a)
        idx = jnp.min(jnp.where(a == m, iota, n_tok))
        toks_out[0, pl.ds(k, 1), :] = rt_ref[0, pl.ds(idx, 1), :]
        cent_out[0, pl.ds(k, 1), :] = cent_ref[0, pl.ds(idx, 1), :]
        return jnp.where(iota == idx, -jnp.inf, a)

    jax.lax.fori_loop(0, _K, body, a)


def _pdist_k(x):
    d2 = jnp.zeros((x.shape[0], x.shape[0]), jnp.float32)
    for c in range(3):
        col = x[:, c:c + 1]
        diff = col - jnp.transpose(col)
        d2 = d2 + diff * diff
    return jnp.sqrt(d2 + 1e-8)


def _triplet_kernel(anc_idx, pos_idx, neg_idx,
                    anc_t_ref, pos_t_ref, neg_t_ref,
                    anc_c_ref, pos_c_ref, neg_c_ref,
                    win_ref, bin_ref, w1_ref, b1_ref, w2_ref, b2_ref,
                    out_ref):
    del anc_idx, pos_idx, neg_idx
    k = _K
    c_dim = win_ref.shape[0]
    inv_sqrt_c = 1.0 / jnp.sqrt(jnp.float32(c_dim))
    ii = jax.lax.broadcasted_iota(jnp.int32, (k, k), 0)
    jj = jax.lax.broadcasted_iota(jnp.int32, (k, k), 1)
    jj_f = jj.astype(jnp.float32)

    anc_rt = jnp.dot(anc_t_ref[0], win_ref[...],
                     preferred_element_type=jnp.float32) + bin_ref[...]
    d_anc = _pdist_k(anc_c_ref[0])

    scores = []
    for n, (t_ref, c_ref) in enumerate(((pos_t_ref, pos_c_ref),
                                        (neg_t_ref, neg_c_ref))):
        nn_rt = jnp.dot(t_ref[0], win_ref[...],
                        preferred_element_type=jnp.float32) + bin_ref[...]
        sim = jax.lax.dot_general(
            anc_rt, nn_rt, (((1,), (1,)), ((), ())),
            preferred_element_type=jnp.float32) * inv_sqrt_c
        sim = sim - jnp.max(sim, axis=1, keepdims=True)
        e = jnp.exp(sim)
        attn = e / jnp.sum(e, axis=1, keepdims=True)
        matched = jnp.dot(attn, c_ref[0],
                          preferred_element_type=jnp.float32)
        diff = d_anc - _pdist_k(matched)
        m_adj = jnp.maximum(0.0, 1.0 - diff * diff / _D_THRESH2)
        m_adj = jnp.where(ii == jj, 0.0, m_adj)

        v = jnp.full((1, k), 1.0 / jnp.sqrt(jnp.float32(k)), jnp.float32)
        for _ in range(_POWER_ITERS):
            v = jnp.dot(v, m_adj, preferred_element_type=jnp.float32)
            v = v / (jnp.sqrt(jnp.sum(v * v)) + 1e-8)

        row = jnp.broadcast_to(v, (k, k))
        col = jnp.transpose(v)
        gt = (row > col) | ((row == col) & (jj < ii))
        rank = jnp.sum(gt.astype(jnp.float32), axis=1, keepdims=True)
        sel = (rank == jj_f).astype(jnp.float32)
        v_sorted = jnp.dot(v, sel, preferred_element_type=jnp.float32)

        h = jnp.maximum(
            jnp.dot(v_sorted, w1_ref[...],
                    preferred_element_type=jnp.float32) + b1_ref[...], 0.0)
        z = jnp.sum(h * w2_ref[...]) + b2_ref[0, 0]
        scores.append(1.0 / (1.0 + jnp.exp(-z)))

    lane = jax.lax.broadcasted_iota(jnp.int32, (1, 2), 1)
    out_ref[0] = jnp.where(lane == 0, scores[0], scores[1])


def kernel(rt, rt_cls_attn, rt_centroids, anc_indices, pos_indices,
           neg_indices, W_in, b_in, W1, b1, W2, b2):
    b_dim, n_tok, c_dim = rt.shape
    t_dim = anc_indices.shape[0]
    k = _K

    if n_tok % 128 == 0:
        r_dim, l_dim = n_tok // 128, 128
    else:
        r_dim, l_dim = 1, n_tok
    attn = rt_cls_attn[..., 0].reshape(b_dim, r_dim, l_dim)
    toks_sel, cent_sel = pl.pallas_call(
        _topk_gather_kernel,
        grid=(b_dim,),
        in_specs=[
            pl.BlockSpec((1, r_dim, l_dim), lambda b: (b, 0, 0)),
            pl.BlockSpec((1, n_tok, c_dim), lambda b: (b, 0, 0)),
            pl.BlockSpec((1, n_tok, 3), lambda b: (b, 0, 0)),
        ],
        out_specs=[
            pl.BlockSpec((1, k, c_dim), lambda b: (b, 0, 0)),
            pl.BlockSpec((1, k, 3), lambda b: (b, 0, 0)),
        ],
        out_shape=[
            jax.ShapeDtypeStruct((b_dim, k, c_dim), jnp.float32),
            jax.ShapeDtypeStruct((b_dim, k, 3), jnp.float32),
        ],
    )(attn, rt, rt_centroids)

    anc_i = jnp.asarray(anc_indices, jnp.int32)
    pos_i = jnp.asarray(pos_indices, jnp.int32)
    neg_i = jnp.asarray(neg_indices, jnp.int32)

    grid_spec = pltpu.PrefetchScalarGridSpec(
        num_scalar_prefetch=3,
        grid=(t_dim,),
        in_specs=[
            pl.BlockSpec((1, k, c_dim), lambda t, a, p, n: (a[t], 0, 0)),
            pl.BlockSpec((1, k, c_dim), lambda t, a, p, n: (p[t], 0, 0)),
            pl.BlockSpec((1, k, c_dim), lambda t, a, p, n: (n[t], 0, 0)),
            pl.BlockSpec((1, k, 3), lambda t, a, p, n: (a[t], 0, 0)),
            pl.BlockSpec((1, k, 3), lambda t, a, p, n: (p[t], 0, 0)),
            pl.BlockSpec((1, k, 3), lambda t, a, p, n: (n[t], 0, 0)),
            pl.BlockSpec((c_dim, c_dim), lambda t, a, p, n: (0, 0)),
            pl.BlockSpec((1, c_dim), lambda t, a, p, n: (0, 0)),
            pl.BlockSpec((k, k), lambda t, a, p, n: (0, 0)),
            pl.BlockSpec((1, k), lambda t, a, p, n: (0, 0)),
            pl.BlockSpec((1, k), lambda t, a, p, n: (0, 0)),
            pl.BlockSpec((1, 1), lambda t, a, p, n: (0, 0)),
        ],
        out_specs=pl.BlockSpec((1, 1, 2), lambda t, a, p, n: (t, 0, 0)),
    )
    scores = pl.pallas_call(
        _triplet_kernel,
        grid_spec=grid_spec,
        out_shape=jax.ShapeDtypeStruct((t_dim, 1, 2), jnp.float32),
    )(anc_i, pos_i, neg_i,
      toks_sel, toks_sel, toks_sel, cent_sel, cent_sel, cent_sel,
      W_in, b_in.reshape(1, c_dim), W1, b1.reshape(1, k),
      W2.reshape(1, k), b2.reshape(1, 1))

    rerank_scores = scores.reshape(t_dim, 2, 1)
    targets = jnp.zeros_like(rerank_scores).at[:, 0].set(1.0)
    return rerank_scores, targets
